# baseline (speedup 1.0000x reference)
import functools

import jax
import jax.numpy as jnp
from jax import lax
from jax.experimental import pallas as pl
from jax.experimental.pallas import tpu as pltpu
from jax.experimental.pallas import tpu_sc as plsc

D_MODEL = 1024
NUM_EXPERT = 8
TOP_K = 2
N_TOKENS = 32768
NOISE_EPS = 0.01

NUM_WORKERS = 32
CHUNK = N_TOKENS // NUM_WORKERS
LANES = 16
GROUPS = CHUNK // LANES

_INV_SQRT2 = 0.7071067811865476
_P = 0.3275911
_A1 = 0.254829592
_A2 = -0.284496736
_A3 = 1.421413741
_A4 = -1.453152027
_A5 = 1.061405429


def _dense_body(wct_ref, inp_ref, noise_ref, slab_ref):
    logits = lax.dot_general(
        wct_ref[...], inp_ref[...],
        dimension_numbers=(((1,), (1,)), ((), ())),
        preferred_element_type=jnp.float32)
    clean = logits[:NUM_EXPERT, :]
    raw = logits[NUM_EXPERT:, :]
    stddev = jax.nn.softplus(raw) + NOISE_EPS
    noisy = clean + noise_ref[...].T * stddev
    slab_ref[...] = jnp.concatenate([clean, noisy, stddev], axis=0)[None]


def _cdf(z):
    x = z * _INV_SQRT2
    ax = jnp.abs(x)
    t = 1.0 / (1.0 + _P * ax)
    poly = ((((_A5 * t + _A4) * t + _A3) * t + _A2) * t + _A1) * t
    erf_abs = 1.0 - poly * jnp.exp(-ax * ax)
    erf = jnp.where(x >= 0.0, erf_abs, -erf_abs)
    return 0.5 * (1.0 + erf)


def _route_body(slab_hbm, idx_hbm, gate_hbm, part_hbm,
                slab_v, idx_v, gate_v, part_v):
    wid = lax.axis_index("c") * 16 + lax.axis_index("s")
    pltpu.sync_copy(slab_hbm.at[wid], slab_v)

    lane = lax.broadcasted_iota(jnp.int32, (LANES,), 0)
    zeros = jnp.zeros((LANES,), jnp.float32)
    neg = jnp.full((LANES,), -jnp.inf, jnp.float32)

    def body(g, accs):
        sl = pl.ds(g * LANES, LANES)
        clean = [slab_v[e, sl] for e in range(NUM_EXPERT)]
        noisy = [slab_v[NUM_EXPERT + e, sl] for e in range(NUM_EXPERT)]
        std = [slab_v[2 * NUM_EXPERT + e, sl] for e in range(NUM_EXPERT)]

        big = jnp.full((LANES,), NUM_EXPERT, jnp.int32)

        def argtop(vals):
            v = vals[0]
            for e in range(1, NUM_EXPERT):
                v = jnp.maximum(v, vals[e])
            ix = big
            for e in range(NUM_EXPERT - 1, -1, -1):
                ix = jnp.where(vals[e] == v, jnp.full((LANES,), e, jnp.int32), ix)
            return v, ix

        v1, i1 = argtop(noisy)
        m2 = [jnp.where(i1 == e, neg, noisy[e]) for e in range(NUM_EXPERT)]
        v2, i2 = argtop(m2)
        m3 = [jnp.where(i2 == e, neg, m2[e]) for e in range(NUM_EXPERT)]
        v3 = m3[0]
        for e in range(1, NUM_EXPERT):
            v3 = jnp.maximum(v3, m3[e])

        a = jnp.exp(v2 - v1)
        g1 = 1.0 / (1.0 + a)
        g2 = 1.0 - g1

        sl_out = pl.ds(g * LANES, LANES)
        idx_v[0, sl_out] = i1
        idx_v[1, sl_out] = i2
        gate_v[0, sl_out] = g1
        gate_v[1, sl_out] = g2

        out = []
        for e in range(NUM_EXPERT):
            inv_std = 1.0 / std[e]
            thr = jnp.where(noisy[e] > v3, v3, v2)
            prob = _cdf((clean[e] - thr) * inv_std)
            imp_e = jnp.where(i1 == e, g1, zeros) + jnp.where(i2 == e, g2, zeros)
            out.append(accs[e] + imp_e)
            out.append(accs[NUM_EXPERT + e] + prob)
        return tuple(out[0::2] + out[1::2])

    init = tuple(zeros for _ in range(2 * NUM_EXPERT))
    accs = lax.fori_loop(0, GROUPS, body, init)

    part = zeros
    for e in range(2 * NUM_EXPERT):
        s = lax.reduce_sum_p.bind(accs[e], axes=(0,))
        part = part + jnp.where(lane == e, jnp.full((LANES,), 1.0) * s, zeros)
    part_v[...] = part

    pltpu.sync_copy(idx_v, idx_hbm.at[wid])
    pltpu.sync_copy(gate_v, gate_hbm.at[wid])
    pltpu.sync_copy(part_v, part_hbm.at[wid])


def _loss_body(part_ref, loss_ref):
    tot = jnp.sum(part_ref[...], axis=0, keepdims=True)
    imp = tot[:, :NUM_EXPERT]
    load = tot[:, NUM_EXPERT:]

    def cv_sq(x):
        mean = jnp.mean(x, keepdims=True)
        var = jnp.sum((x - mean) ** 2, keepdims=True) / (NUM_EXPERT - 1)
        return var / (mean * mean + 1e-10)

    loss_ref[...] = cv_sq(imp) + cv_sq(load)


@jax.jit
def kernel(inp, w_gate, w_noise, noise):
    wct = jnp.concatenate([w_gate, w_noise], axis=1).T

    slabs = pl.pallas_call(
        _dense_body,
        grid=(NUM_WORKERS,),
        in_specs=[
            pl.BlockSpec((2 * NUM_EXPERT, D_MODEL), lambda i: (0, 0)),
            pl.BlockSpec((CHUNK, D_MODEL), lambda i: (i, 0)),
            pl.BlockSpec((CHUNK, NUM_EXPERT), lambda i: (i, 0)),
        ],
        out_specs=pl.BlockSpec((1, 3 * NUM_EXPERT, CHUNK), lambda i: (i, 0, 0)),
        out_shape=jax.ShapeDtypeStruct((NUM_WORKERS, 3 * NUM_EXPERT, CHUNK),
                                       jnp.float32),
    )(wct, inp, noise)

    mesh = plsc.VectorSubcoreMesh(core_axis_name="c", subcore_axis_name="s")
    route = functools.partial(
        pl.kernel,
        mesh=mesh,
        compiler_params=pltpu.CompilerParams(needs_layout_passes=False),
        out_type=[
            jax.ShapeDtypeStruct((NUM_WORKERS, TOP_K, CHUNK), jnp.int32),
            jax.ShapeDtypeStruct((NUM_WORKERS, TOP_K, CHUNK), jnp.float32),
            jax.ShapeDtypeStruct((NUM_WORKERS, LANES), jnp.float32),
        ],
        scratch_types=[
            pltpu.VMEM((3 * NUM_EXPERT, CHUNK), jnp.float32),
            pltpu.VMEM((TOP_K, CHUNK), jnp.int32),
            pltpu.VMEM((TOP_K, CHUNK), jnp.float32),
            pltpu.VMEM((LANES,), jnp.float32),
        ],
    )(_route_body)
    idx32, gates32, parts = route(slabs)

    loss = pl.pallas_call(
        _loss_body,
        out_shape=jax.ShapeDtypeStruct((1, 1), jnp.float32),
    )(parts)

    top_k_indices = idx32.transpose(0, 2, 1).reshape(-1)
    top_k_gates = gates32.transpose(0, 2, 1).reshape(N_TOKENS, 1, TOP_K)
    return top_k_indices, top_k_gates, loss.reshape(())
